# baseline (speedup 1.0000x reference)
import jax
import jax.numpy as jnp
from jax.experimental import pallas as pl

B = 16384
LT = 50
LP = 200
V = 20
D = 64
H = 128
CB = 2048


def _counts_t(tok):
    l = tok.shape[0]
    nfull = l // 16
    one = jnp.ones((), jnp.int16)
    zero = jnp.zeros((), jnp.int16)
    rows = []
    for v in range(V):
        m = jnp.where(tok == jnp.int16(v), one, zero)
        acc = m[0:16]
        for t in range(1, nfull):
            acc = acc + m[16 * t:16 * (t + 1)]
        cnt = jnp.sum(acc.astype(jnp.float32), axis=0, keepdims=True)
        if l % 16:
            rem = m[16 * nfull:l]
            cnt = cnt + jnp.sum(rem.astype(jnp.float32), axis=0,
                                keepdims=True)
        rows.append(cnt)
    return jnp.concatenate(rows, axis=0)


def _body(tcr_ref, lt_ref, pmhc_ref, lp_ref, embed_ref, w1_ref, b1_ref,
          w2_ref, b2_ref, out_ref):
    embed = embed_ref[:, :]
    w1 = w1_ref[:, :]
    dn = (((1,), (1,)), ((), ()))
    hi = jax.lax.Precision.HIGHEST
    e1a = jax.lax.dot_general(embed, w1[:, :D], dn, precision=hi,
                              preferred_element_type=jnp.float32)
    e1b = jax.lax.dot_general(embed, w1[:, D:], dn, precision=hi,
                              preferred_element_type=jnp.float32)

    tcr_t = tcr_ref[:, :].astype(jnp.int16)
    pmhc_t = pmhc_ref[:, :].astype(jnp.int16)

    inv_lt = jnp.reshape(1.0 / lt_ref[:], (1, CB))
    inv_lp = jnp.reshape(1.0 / lp_ref[:], (1, CB))
    ct = _counts_t(tcr_t) * inv_lt
    cp = _counts_t(pmhc_t) * inv_lp

    dnt = (((0,), (0,)), ((), ()))
    h = (jax.lax.dot_general(ct, e1a, dnt, precision=hi,
                             preferred_element_type=jnp.float32)
         + jax.lax.dot_general(cp, e1b, dnt, precision=hi,
                               preferred_element_type=jnp.float32)
         + jnp.reshape(b1_ref[:], (1, H)))
    h = jnp.maximum(h, 0.0)
    out = jnp.sum(h * w2_ref[:, :], axis=1) + b2_ref[0]
    out_ref[:] = out


def kernel(tcr, tcr_len, pmhc, pmhc_len, embed, W1, b1, W2, b2):
    grid = (B // CB,)
    out = pl.pallas_call(
        _body,
        grid=grid,
        in_specs=[
            pl.BlockSpec((LT, CB), lambda i: (0, i)),
            pl.BlockSpec((CB,), lambda i: (i,)),
            pl.BlockSpec((LP, CB), lambda i: (0, i)),
            pl.BlockSpec((CB,), lambda i: (i,)),
            pl.BlockSpec((V, D), lambda i: (0, 0)),
            pl.BlockSpec((H, 2 * D), lambda i: (0, 0)),
            pl.BlockSpec((H,), lambda i: (0,)),
            pl.BlockSpec((1, H), lambda i: (0, 0)),
            pl.BlockSpec((1,), lambda i: (0,)),
        ],
        out_specs=pl.BlockSpec((CB,), lambda i: (i,)),
        out_shape=jax.ShapeDtypeStruct((B,), jnp.float32),
    )(tcr.T, tcr_len, pmhc.T, pmhc_len, embed, W1, b1, W2, b2)
    return out


# speedup vs baseline: 2.0837x; 2.0837x over previous
import jax
import jax.numpy as jnp
from jax.experimental import pallas as pl

B = 16384
LT = 50
LP = 200
V = 20
D = 64
H = 128
CB = 2048
SENTINEL = 255


def _counts_t(tok16, lpad):
    l = tok16.shape[0]
    if lpad > l:
        tok16 = jnp.concatenate(
            [tok16, jnp.full((lpad - l, CB), SENTINEL, jnp.int16)], axis=0)
    ntile = lpad // 16
    one = jnp.ones((), jnp.int16)
    zero = jnp.zeros((), jnp.int16)
    rows = []
    for v in range(V):
        m = jnp.where(tok16 == jnp.int16(v), one, zero)
        tiles = [m[16 * t:16 * (t + 1)] for t in range(ntile)]
        while len(tiles) > 1:
            tiles = [a + b for a, b in zip(tiles[::2], tiles[1::2])] + (
                [tiles[-1]] if len(tiles) % 2 else [])
        cnt = jnp.sum(tiles[0], axis=0, keepdims=True)
        rows.append(cnt.astype(jnp.float32))
    return jnp.concatenate(rows, axis=0).astype(jnp.bfloat16)


def _split_dot_t(table, cnt_bf):
    t_hi = table.astype(jnp.bfloat16)
    t_lo = (table - t_hi.astype(jnp.float32)).astype(jnp.bfloat16)
    dn = (((0,), (0,)), ((), ()))
    return (jax.lax.dot_general(t_hi, cnt_bf, dn,
                                preferred_element_type=jnp.float32)
            + jax.lax.dot_general(t_lo, cnt_bf, dn,
                                  preferred_element_type=jnp.float32))


def _body(tcr_ref, lt_ref, pmhc_ref, lp_ref, embed_ref, w1_ref, b1_ref,
          w2_ref, b2_ref, out_ref):
    embed = embed_ref[:, :]
    w1 = w1_ref[:, :]
    dn = (((1,), (1,)), ((), ()))
    hi = jax.lax.Precision.HIGHEST
    e1a = jax.lax.dot_general(embed, w1[:, :D], dn, precision=hi,
                              preferred_element_type=jnp.float32)
    e1b = jax.lax.dot_general(embed, w1[:, D:], dn, precision=hi,
                              preferred_element_type=jnp.float32)

    ct = _counts_t(tcr_ref[:, :].astype(jnp.int16), 64)
    cp = _counts_t(pmhc_ref[:, :].astype(jnp.int16), 208)

    inv_lt = jnp.reshape(1.0 / lt_ref[:], (1, CB))
    inv_lp = jnp.reshape(1.0 / lp_ref[:], (1, CB))

    b1_col = jnp.transpose(jnp.reshape(b1_ref[:], (1, H)))
    w2_col = jnp.transpose(w2_ref[:, :])

    h = (_split_dot_t(e1a, ct) * inv_lt
         + _split_dot_t(e1b, cp) * inv_lp
         + b1_col)
    h = jnp.maximum(h, 0.0)
    out_ref[:] = jnp.sum(h * w2_col, axis=0) + b2_ref[0]


def kernel(tcr, tcr_len, pmhc, pmhc_len, embed, W1, b1, W2, b2):
    grid = (B // CB,)
    out = pl.pallas_call(
        _body,
        grid=grid,
        in_specs=[
            pl.BlockSpec((LT, CB), lambda i: (0, i)),
            pl.BlockSpec((CB,), lambda i: (i,)),
            pl.BlockSpec((LP, CB), lambda i: (0, i)),
            pl.BlockSpec((CB,), lambda i: (i,)),
            pl.BlockSpec((V, D), lambda i: (0, 0)),
            pl.BlockSpec((H, 2 * D), lambda i: (0, 0)),
            pl.BlockSpec((H,), lambda i: (0,)),
            pl.BlockSpec((1, H), lambda i: (0, 0)),
            pl.BlockSpec((1,), lambda i: (0,)),
        ],
        out_specs=pl.BlockSpec((CB,), lambda i: (i,)),
        out_shape=jax.ShapeDtypeStruct((B,), jnp.float32),
    )(tcr.T, tcr_len, pmhc.T, pmhc_len, embed, W1, b1, W2, b2)
    return out


# speedup vs baseline: 2.1431x; 1.0285x over previous
import jax
import jax.numpy as jnp
from jax.experimental import pallas as pl

B = 16384
LT = 50
LP = 200
V = 20
D = 64
H = 128
CB = 4096
SENTINEL = 255


def _counts_t(tok16, lpad):
    l = tok16.shape[0]
    if lpad > l:
        tok16 = jnp.concatenate(
            [tok16, jnp.full((lpad - l, CB), SENTINEL, jnp.int16)], axis=0)
    ntile = lpad // 16
    one = jnp.ones((), jnp.int16)
    zero = jnp.zeros((), jnp.int16)
    rows = []
    for v in range(V):
        m = jnp.where(tok16 == jnp.int16(v), one, zero)
        tiles = [m[16 * t:16 * (t + 1)] for t in range(ntile)]
        while len(tiles) > 1:
            tiles = [a + b for a, b in zip(tiles[::2], tiles[1::2])] + (
                [tiles[-1]] if len(tiles) % 2 else [])
        cnt = jnp.sum(tiles[0], axis=0, keepdims=True)
        rows.append(cnt.astype(jnp.float32))
    return jnp.concatenate(rows, axis=0).astype(jnp.bfloat16)


def _split_dot_t(table, cnt_bf):
    t_hi = table.astype(jnp.bfloat16)
    t_lo = (table - t_hi.astype(jnp.float32)).astype(jnp.bfloat16)
    dn = (((0,), (0,)), ((), ()))
    return (jax.lax.dot_general(t_hi, cnt_bf, dn,
                                preferred_element_type=jnp.float32)
            + jax.lax.dot_general(t_lo, cnt_bf, dn,
                                  preferred_element_type=jnp.float32))


def _body(tcr_ref, lt_ref, pmhc_ref, lp_ref, embed_ref, w1_ref, b1_ref,
          w2_ref, b2_ref, out_ref):
    embed = embed_ref[:, :]
    w1 = w1_ref[:, :]
    dn = (((1,), (1,)), ((), ()))
    hi = jax.lax.Precision.HIGHEST
    e1a = jax.lax.dot_general(embed, w1[:, :D], dn, precision=hi,
                              preferred_element_type=jnp.float32)
    e1b = jax.lax.dot_general(embed, w1[:, D:], dn, precision=hi,
                              preferred_element_type=jnp.float32)

    ct = _counts_t(tcr_ref[:, :].astype(jnp.int16), 64)
    cp = _counts_t(pmhc_ref[:, :].astype(jnp.int16), 208)

    inv_lt = jnp.reshape(1.0 / lt_ref[:], (1, CB))
    inv_lp = jnp.reshape(1.0 / lp_ref[:], (1, CB))

    b1_col = jnp.transpose(jnp.reshape(b1_ref[:], (1, H)))
    w2_col = jnp.transpose(w2_ref[:, :])

    h = (_split_dot_t(e1a, ct) * inv_lt
         + _split_dot_t(e1b, cp) * inv_lp
         + b1_col)
    h = jnp.maximum(h, 0.0)
    out_ref[:] = jnp.sum(h * w2_col, axis=0) + b2_ref[0]


def kernel(tcr, tcr_len, pmhc, pmhc_len, embed, W1, b1, W2, b2):
    grid = (B // CB,)
    out = pl.pallas_call(
        _body,
        grid=grid,
        in_specs=[
            pl.BlockSpec((LT, CB), lambda i: (0, i)),
            pl.BlockSpec((CB,), lambda i: (i,)),
            pl.BlockSpec((LP, CB), lambda i: (0, i)),
            pl.BlockSpec((CB,), lambda i: (i,)),
            pl.BlockSpec((V, D), lambda i: (0, 0)),
            pl.BlockSpec((H, 2 * D), lambda i: (0, 0)),
            pl.BlockSpec((H,), lambda i: (0,)),
            pl.BlockSpec((1, H), lambda i: (0, 0)),
            pl.BlockSpec((1,), lambda i: (0,)),
        ],
        out_specs=pl.BlockSpec((CB,), lambda i: (i,)),
        out_shape=jax.ShapeDtypeStruct((B,), jnp.float32),
    )(tcr.T, tcr_len, pmhc.T, pmhc_len, embed, W1, b1, W2, b2)
    return out
